# Initial kernel scaffold; baseline (speedup 1.0000x reference)
import jax
import jax.numpy as jnp
from jax.experimental import pallas as pl


def kernel(net_grad, loss_per_pix, prev_samples, cdf, noise, rand_ten, const_img_id):
    raise NotImplementedError("write your pallas kernel here")


# speedup vs baseline: 21.3191x; 21.3191x over previous
import functools

import jax
import jax.numpy as jnp
from jax import lax
from jax.experimental import pallas as pl
from jax.experimental.pallas import tpu as pltpu
from jax.experimental.pallas import tpu_sc as plsc

N_IMGS = 100
HEIGHT = 512
WIDTH = 512
N_RAYS = 65536
CDF_W = HEIGHT * WIDTH + 1
S_PER_IMG = N_RAYS // N_IMGS
U_NUM = int(0.1 * N_RAYS)
K_TH = U_NUM + 1
ROWS = 512
LANES = 128
NC = 2
NS = 16
NW = NC * NS
CHUNK = N_RAYS // NW
NG = CHUNK // 16
SEARCH_STEPS = 18


def _lane_cumsum(x):
    for sh in (1, 2, 4, 8, 16, 32, 64):
        x = x + jnp.concatenate(
            [jnp.zeros((x.shape[0], sh), x.dtype), x[:, :-sh]], axis=1)
    return x


def _row_cumsum(x):
    sh = 1
    while sh < x.shape[0]:
        x = x + jnp.concatenate(
            [jnp.zeros((sh, 1), x.dtype), x[:-sh, :]], axis=0)
        sh *= 2
    return x


def _tc_body(loss_ref, p0_ref, p1_ref, id_ref,
             mask_ref, fi_ref, cnt_smem, cum_smem):
    loss = loss_ref[...]
    p0 = p0_ref[...]
    p1 = p1_ref[...]

    bits = lax.bitcast_convert_type(loss, jnp.int32)
    flip = jnp.int32(0x7FFFFFFF)
    key = jnp.where(bits < 0, bits ^ flip, bits)

    def bs_body(_, carry):
        lo, hi = carry
        mid = (lo & hi) + ((lo ^ hi) >> 1)
        c = jnp.sum((key <= mid).astype(jnp.int32))
        ge = c >= K_TH
        return (jnp.where(ge, lo, mid + 1), jnp.where(ge, mid, hi))

    lo0 = jnp.int32(-2147483647 - 1)
    hi0 = jnp.int32(2147483647)
    _, vfin = lax.fori_loop(0, 32, bs_body, (lo0, hi0))
    thr_bits = jnp.where(vfin < 0, vfin ^ flip, vfin)
    thr = lax.bitcast_convert_type(thr_bits, jnp.float32)

    oob = (p0 < 0.0) | (p0 > 1.0) | (p1 < 0.0) | (p1 > 1.0)
    maskb = oob | (loss <= thr)
    m = maskb.astype(jnp.int32)
    mask_ref[...] = m

    ids = id_ref[...]

    def cnt_body(b, carry):
        cnt_smem[b] = jnp.sum(jnp.where(ids == b, m, 0))
        return carry

    lax.fori_loop(0, N_IMGS, cnt_body, 0)

    def cum_body(b, acc):
        acc = acc + cnt_smem[b]
        cum_smem[b] = acc
        return acc

    lax.fori_loop(0, N_IMGS, cum_body, jnp.int32(0))

    c1 = _lane_cumsum(m)
    rowsum = c1[:, LANES - 1:LANES]
    excl = _row_cumsum(rowsum) - rowsum
    rank = excl + c1 - 1

    zeros = jnp.zeros((ROWS, LANES), jnp.int32)

    def img_body(b, carry):
        img, ce = carry
        cb = cum_smem[b]
        le = cb <= rank
        return (img + le.astype(jnp.int32), jnp.where(le, cb, ce))

    img, ce = lax.fori_loop(0, N_IMGS, img_body, (zeros, zeros))
    img = jnp.minimum(img, N_IMGS - 1)
    within = jnp.clip(rank - ce, 0, S_PER_IMG - 1)
    fi_ref[...] = img * S_PER_IMG + within


def _tc_stage(loss2, p0, p1, id2):
    i32 = jnp.int32
    return pl.pallas_call(
        _tc_body,
        out_shape=[
            jax.ShapeDtypeStruct((ROWS, LANES), i32),
            jax.ShapeDtypeStruct((ROWS, LANES), i32),
        ],
        scratch_shapes=[
            pltpu.SMEM((N_IMGS,), i32),
            pltpu.SMEM((N_IMGS,), i32),
        ],
    )(loss2, p0, p1, id2)


def _mesh():
    return plsc.VectorSubcoreMesh(core_axis_name="c", subcore_axis_name="s")


def _worker_base():
    wid = lax.axis_index("s") * NC + lax.axis_index("c")
    return wid * CHUNK


def _sc_search_body(u_hbm, cdf_hbm, offs_hbm, out_hbm,
                    u_v, lo_v, hi_v, mid_v, idx_v, val_v, off_v, sem):
    base = _worker_base()
    pltpu.sync_copy(u_hbm.at[pl.ds(base, CHUNK)], u_v)
    pltpu.sync_copy(offs_hbm.at[pl.ds(base, CHUNK)], off_v)

    def init_body(i, carry):
        sl = pl.ds(i * 16, 16)
        z = off_v[sl]
        lo_v[sl] = z ^ z
        hi_v[sl] = (z ^ z) + CDF_W
        return carry

    lax.fori_loop(0, NG, init_body, 0)

    def step(_, carry):
        def mid_body(i, c2):
            sl = pl.ds(i * 16, 16)
            lo = lo_v[sl]
            hi = hi_v[sl]
            mid = (lo + hi) >> 1
            mid_v[sl] = mid
            idx_v[sl] = jnp.minimum(off_v[sl] + mid,
                                    jnp.int32(N_IMGS * CDF_W - 1))
            return c2

        lax.fori_loop(0, NG, mid_body, 0)

        copies = [
            pltpu.async_copy(cdf_hbm.at[idx_v.at[pl.ds(j * 128, 128)]],
                             val_v.at[pl.ds(j * 128, 128)], sem)
            for j in range(16)
        ]
        for cp in copies:
            cp.wait()

        def upd_body(i, c2):
            sl = pl.ds(i * 16, 16)
            lo = lo_v[sl]
            hi = hi_v[sl]
            mid = mid_v[sl]
            act = lo < hi
            cond = val_v[sl] <= u_v[sl]
            gt = val_v[sl] > u_v[sl]
            lo_v[sl] = jnp.where(act & cond, mid + 1, lo)
            hi_v[sl] = jnp.where(act & gt, mid, hi)
            return c2

        lax.fori_loop(0, NG, upd_body, 0)
        return carry

    lax.fori_loop(0, SEARCH_STEPS, step, 0)

    def fin_body(i, carry):
        sl = pl.ds(i * 16, 16)
        idx_v[sl] = jnp.clip(lo_v[sl] - 1, 0, HEIGHT * WIDTH - 1)
        return carry

    lax.fori_loop(0, NG, fin_body, 0)
    pltpu.sync_copy(idx_v, out_hbm.at[pl.ds(base, CHUNK)])


@functools.cache
def _sc_search_kernel():
    @functools.partial(
        pl.kernel,
        mesh=_mesh(),
        out_type=jax.ShapeDtypeStruct((N_RAYS,), jnp.int32),
        scratch_types=[
            pltpu.VMEM((CHUNK,), jnp.float32),
            pltpu.VMEM((CHUNK,), jnp.int32),
            pltpu.VMEM((CHUNK,), jnp.int32),
            pltpu.VMEM((CHUNK,), jnp.int32),
            pltpu.VMEM((CHUNK,), jnp.int32),
            pltpu.VMEM((CHUNK,), jnp.float32),
            pltpu.VMEM((CHUNK,), jnp.int32),
            pltpu.SemaphoreType.DMA,
        ],
    )
    def _sc_search(u_hbm, cdf_hbm, offs_hbm, out_hbm, *scratch):
        _sc_search_body(u_hbm, cdf_hbm, offs_hbm, out_hbm, *scratch)

    return _sc_search


def _rne(x):
    t = x + 0.5
    ti = t.astype(jnp.int32)
    f = ti.astype(jnp.float32)
    tie = f == t
    odd = (ti & 1) == 1
    return f - jnp.where(tie & odd, 1.0, 0.0)


def _sc_gather_body(b1_hbm, fi_hbm, mask_hbm, q0_hbm, q1_hbm,
                    o0_hbm, o1_hbm,
                    fi_v, g_v, m_v, q0_v, q1_v, o0_v, o1_v, sem):
    base = _worker_base()
    sl_all = pl.ds(base, CHUNK)
    pltpu.sync_copy(fi_hbm.at[sl_all], fi_v)
    copies = [
        pltpu.async_copy(b1_hbm.at[fi_v.at[pl.ds(j * 128, 128)]],
                         g_v.at[pl.ds(j * 128, 128)], sem)
        for j in range(16)
    ]
    pltpu.sync_copy(mask_hbm.at[sl_all], m_v)
    pltpu.sync_copy(q0_hbm.at[sl_all], q0_v)
    pltpu.sync_copy(q1_hbm.at[sl_all], q1_v)
    for cp in copies:
        cp.wait()

    def body(i, carry):
        sl = pl.ds(i * 16, 16)
        b1 = g_v[sl]
        n0 = (b1 >> 9).astype(jnp.float32) / 511.0
        n1 = (b1 & 511).astype(jnp.float32) / 511.0
        mk = m_v[sl] != 0
        x0 = jnp.clip(jnp.where(mk, n0, q0_v[sl]), 0.0, 1.0) * 511.0
        x1 = jnp.clip(jnp.where(mk, n1, q1_v[sl]), 0.0, 1.0) * 511.0
        o0_v[sl] = _rne(x0)
        o1_v[sl] = _rne(x1)
        return carry

    lax.fori_loop(0, NG, body, 0)
    pltpu.sync_copy(o0_v, o0_hbm.at[sl_all])
    pltpu.sync_copy(o1_v, o1_hbm.at[sl_all])


@functools.cache
def _sc_gather_kernel():
    @functools.partial(
        pl.kernel,
        mesh=_mesh(),
        out_type=[
            jax.ShapeDtypeStruct((N_RAYS,), jnp.float32),
            jax.ShapeDtypeStruct((N_RAYS,), jnp.float32),
        ],
        scratch_types=[
            pltpu.VMEM((CHUNK,), jnp.int32),
            pltpu.VMEM((CHUNK,), jnp.int32),
            pltpu.VMEM((CHUNK,), jnp.int32),
            pltpu.VMEM((CHUNK,), jnp.float32),
            pltpu.VMEM((CHUNK,), jnp.float32),
            pltpu.VMEM((CHUNK,), jnp.float32),
            pltpu.VMEM((CHUNK,), jnp.float32),
            pltpu.SemaphoreType.DMA,
        ],
    )
    def _sc_gather(b1_hbm, fi_hbm, mask_hbm, q0_hbm, q1_hbm, *rest):
        _sc_gather_body(b1_hbm, fi_hbm, mask_hbm, q0_hbm, q1_hbm, *rest)

    return _sc_gather


def kernel(net_grad, loss_per_pix, prev_samples, cdf, noise, rand_ten,
           const_img_id):
    g = net_grad * 20.0 + noise * 0.02
    p = prev_samples + g
    q = p.at[-U_NUM:].set(rand_ten[-U_NUM:])

    loss2 = loss_per_pix.reshape(ROWS, LANES)
    p0 = p[:, 0].reshape(ROWS, LANES)
    p1 = p[:, 1].reshape(ROWS, LANES)
    id2 = const_img_id.astype(jnp.int32).reshape(ROWS, LANES)

    mask2, fi2 = _tc_stage(loss2, p0, p1, id2)
    q0_2 = q[:, 0]
    q1_2 = q[:, 1]

    u = jax.random.uniform(jax.random.key(1), (N_IMGS, S_PER_IMG),
                           dtype=jnp.float32)
    pad = N_RAYS - N_IMGS * S_PER_IMG
    u_flat = jnp.concatenate(
        [u.reshape(-1), jnp.full((pad,), 0.5, jnp.float32)])

    offs = jnp.minimum(jnp.arange(N_RAYS, dtype=jnp.int32) // S_PER_IMG,
                       N_IMGS - 1) * CDF_W
    b1 = _sc_search_kernel()(u_flat, cdf.reshape(-1), offs)
    o0, o1 = _sc_gather_kernel()(b1, fi2.reshape(-1), mask2.reshape(-1),
                                 q0_2.reshape(-1), q1_2.reshape(-1))
    return jnp.stack([o0, o1], axis=1)


# speedup vs baseline: 21.5433x; 1.0105x over previous
import functools

import jax
import jax.numpy as jnp
from jax import lax
from jax.experimental import pallas as pl
from jax.experimental.pallas import tpu as pltpu
from jax.experimental.pallas import tpu_sc as plsc

N_IMGS = 100
HEIGHT = 512
WIDTH = 512
N_RAYS = 65536
CDF_W = HEIGHT * WIDTH + 1
S_PER_IMG = N_RAYS // N_IMGS
U_NUM = int(0.1 * N_RAYS)
K_TH = U_NUM + 1
ROWS = 512
LANES = 128
NC = 2
NS = 16
NW = NC * NS
CHUNK = N_RAYS // NW
NG = CHUNK // 16
SEARCH_STEPS = 18


def _lane_cumsum(x):
    for sh in (1, 2, 4, 8, 16, 32, 64):
        x = x + jnp.concatenate(
            [jnp.zeros((x.shape[0], sh), x.dtype), x[:, :-sh]], axis=1)
    return x


def _row_cumsum(x):
    sh = 1
    while sh < x.shape[0]:
        x = x + jnp.concatenate(
            [jnp.zeros((sh, 1), x.dtype), x[:-sh, :]], axis=0)
        sh *= 2
    return x


def _tc_body(loss_ref, p0_ref, p1_ref, id_ref,
             mask_ref, fi_ref, cnt_smem, cum_smem):
    loss = loss_ref[...]
    p0 = p0_ref[...]
    p1 = p1_ref[...]

    bits = lax.bitcast_convert_type(loss, jnp.int32)
    flip = jnp.int32(0x7FFFFFFF)
    key = jnp.where(bits < 0, bits ^ flip, bits)

    def bs_body(_, carry):
        lo, hi = carry
        mid = (lo & hi) + ((lo ^ hi) >> 1)
        c = jnp.sum((key <= mid).astype(jnp.int32))
        ge = c >= K_TH
        return (jnp.where(ge, lo, mid + 1), jnp.where(ge, mid, hi))

    lo0 = jnp.int32(-2147483647 - 1)
    hi0 = jnp.int32(2147483647)
    _, vfin = lax.fori_loop(0, 32, bs_body, (lo0, hi0))
    thr_bits = jnp.where(vfin < 0, vfin ^ flip, vfin)
    thr = lax.bitcast_convert_type(thr_bits, jnp.float32)

    oob = (p0 < 0.0) | (p0 > 1.0) | (p1 < 0.0) | (p1 > 1.0)
    maskb = oob | (loss <= thr)
    m = maskb.astype(jnp.int32)
    mask_ref[...] = m

    ids = id_ref[...]

    def cnt_body(b, carry):
        cnt_smem[b] = jnp.sum(jnp.where(ids == b, m, 0))
        return carry

    lax.fori_loop(0, N_IMGS, cnt_body, 0)

    def cum_body(b, acc):
        acc = acc + cnt_smem[b]
        cum_smem[b] = acc
        return acc

    lax.fori_loop(0, N_IMGS, cum_body, jnp.int32(0))

    c1 = _lane_cumsum(m)
    rowsum = c1[:, LANES - 1:LANES]
    excl = _row_cumsum(rowsum) - rowsum
    rank = excl + c1 - 1

    zeros = jnp.zeros((ROWS, LANES), jnp.int32)

    def img_body(b, carry):
        img, ce = carry
        cb = cum_smem[b]
        le = cb <= rank
        return (img + le.astype(jnp.int32), jnp.where(le, cb, ce))

    img, ce = lax.fori_loop(0, N_IMGS, img_body, (zeros, zeros))
    img = jnp.minimum(img, N_IMGS - 1)
    within = jnp.clip(rank - ce, 0, S_PER_IMG - 1)
    fi_ref[...] = img * S_PER_IMG + within


def _tc_stage(loss2, p0, p1, id2):
    i32 = jnp.int32
    return pl.pallas_call(
        _tc_body,
        out_shape=[
            jax.ShapeDtypeStruct((ROWS, LANES), i32),
            jax.ShapeDtypeStruct((ROWS, LANES), i32),
        ],
        scratch_shapes=[
            pltpu.SMEM((N_IMGS,), i32),
            pltpu.SMEM((N_IMGS,), i32),
        ],
    )(loss2, p0, p1, id2)


def _mesh():
    return plsc.VectorSubcoreMesh(core_axis_name="c", subcore_axis_name="s")


def _worker_base():
    wid = lax.axis_index("s") * NC + lax.axis_index("c")
    return wid * CHUNK


def _sc_search_body(u_hbm, cdf_hbm, offs_hbm, out_hbm,
                    u_v, pos_v, idx_v, val_v, off_v, sem):
    base = _worker_base()
    pltpu.sync_copy(u_hbm.at[pl.ds(base, CHUNK)], u_v)
    pltpu.sync_copy(offs_hbm.at[pl.ds(base, CHUNK)], off_v)

    def init_body(i, carry):
        sl = pl.ds(i * 16, 16)
        z = off_v[sl]
        pos_v[sl] = z ^ z
        idx_v[sl] = z + (1 << 17)
        return carry

    lax.fori_loop(0, NG, init_body, 0, unroll=8)

    for r in range(SEARCH_STEPS):
        copies = [
            pltpu.async_copy(cdf_hbm.at[idx_v.at[pl.ds(j * 128, 128)]],
                             val_v.at[pl.ds(j * 128, 128)], sem)
            for j in range(16)
        ]
        for cp in copies:
            cp.wait()

        step = 1 << (SEARCH_STEPS - 1 - r)
        nstep = step >> 1
        last = r == SEARCH_STEPS - 1

        def upd_body(i, c2, _step=step, _nstep=nstep, _last=last):
            sl = pl.ds(i * 16, 16)
            pos = pos_v[sl]
            take = val_v[sl] <= u_v[sl]
            pos = jnp.where(take, pos + _step, pos)
            pos_v[sl] = pos
            if _last:
                idx_v[sl] = pos
            else:
                idx_v[sl] = off_v[sl] + (pos + _nstep)
            return c2

        lax.fori_loop(0, NG, upd_body, 0, unroll=8)

    pltpu.sync_copy(idx_v, out_hbm.at[pl.ds(base, CHUNK)])


@functools.cache
def _sc_search_kernel():
    @functools.partial(
        pl.kernel,
        mesh=_mesh(),
        out_type=jax.ShapeDtypeStruct((N_RAYS,), jnp.int32),
        scratch_types=[
            pltpu.VMEM((CHUNK,), jnp.float32),
            pltpu.VMEM((CHUNK,), jnp.int32),
            pltpu.VMEM((CHUNK,), jnp.int32),
            pltpu.VMEM((CHUNK,), jnp.float32),
            pltpu.VMEM((CHUNK,), jnp.int32),
            pltpu.SemaphoreType.DMA,
        ],
    )
    def _sc_search(u_hbm, cdf_hbm, offs_hbm, out_hbm, *scratch):
        _sc_search_body(u_hbm, cdf_hbm, offs_hbm, out_hbm, *scratch)

    return _sc_search


def _rne(x):
    t = x + 0.5
    ti = t.astype(jnp.int32)
    f = ti.astype(jnp.float32)
    tie = f == t
    odd = (ti & 1) == 1
    return f - jnp.where(tie & odd, 1.0, 0.0)


def _sc_gather_body(b1_hbm, fi_hbm, mask_hbm, q0_hbm, q1_hbm,
                    o0_hbm, o1_hbm,
                    fi_v, g_v, m_v, q0_v, q1_v, o0_v, o1_v, sem):
    base = _worker_base()
    sl_all = pl.ds(base, CHUNK)
    pltpu.sync_copy(fi_hbm.at[sl_all], fi_v)
    copies = [
        pltpu.async_copy(b1_hbm.at[fi_v.at[pl.ds(j * 128, 128)]],
                         g_v.at[pl.ds(j * 128, 128)], sem)
        for j in range(16)
    ]
    pltpu.sync_copy(mask_hbm.at[sl_all], m_v)
    pltpu.sync_copy(q0_hbm.at[sl_all], q0_v)
    pltpu.sync_copy(q1_hbm.at[sl_all], q1_v)
    for cp in copies:
        cp.wait()

    def body(i, carry):
        sl = pl.ds(i * 16, 16)
        b1 = g_v[sl]
        n0 = (b1 >> 9).astype(jnp.float32) / 511.0
        n1 = (b1 & 511).astype(jnp.float32) / 511.0
        mk = m_v[sl] != 0
        x0 = jnp.clip(jnp.where(mk, n0, q0_v[sl]), 0.0, 1.0) * 511.0
        x1 = jnp.clip(jnp.where(mk, n1, q1_v[sl]), 0.0, 1.0) * 511.0
        o0_v[sl] = _rne(x0)
        o1_v[sl] = _rne(x1)
        return carry

    lax.fori_loop(0, NG, body, 0, unroll=8)
    pltpu.sync_copy(o0_v, o0_hbm.at[sl_all])
    pltpu.sync_copy(o1_v, o1_hbm.at[sl_all])


@functools.cache
def _sc_gather_kernel():
    @functools.partial(
        pl.kernel,
        mesh=_mesh(),
        out_type=[
            jax.ShapeDtypeStruct((N_RAYS,), jnp.float32),
            jax.ShapeDtypeStruct((N_RAYS,), jnp.float32),
        ],
        scratch_types=[
            pltpu.VMEM((CHUNK,), jnp.int32),
            pltpu.VMEM((CHUNK,), jnp.int32),
            pltpu.VMEM((CHUNK,), jnp.int32),
            pltpu.VMEM((CHUNK,), jnp.float32),
            pltpu.VMEM((CHUNK,), jnp.float32),
            pltpu.VMEM((CHUNK,), jnp.float32),
            pltpu.VMEM((CHUNK,), jnp.float32),
            pltpu.SemaphoreType.DMA,
        ],
    )
    def _sc_gather(b1_hbm, fi_hbm, mask_hbm, q0_hbm, q1_hbm, *rest):
        _sc_gather_body(b1_hbm, fi_hbm, mask_hbm, q0_hbm, q1_hbm, *rest)

    return _sc_gather


def _np_threefry_uniform(seed, n):
    import numpy as np

    def rotl(x, r):
        return ((x << np.uint32(r)) | (x >> np.uint32(32 - r))).astype(
            np.uint32)

    rot = [[13, 15, 26, 6], [17, 29, 16, 24]]
    k1 = np.uint32(seed >> 32)
    k2 = np.uint32(seed & 0xFFFFFFFF)
    ks = [k1, k2, np.uint32(k1 ^ k2 ^ np.uint32(0x1BD11BDA))]
    x1 = np.full(n, ks[0], np.uint32)
    x2 = (np.arange(n, dtype=np.uint32) + ks[1]).astype(np.uint32)
    for i in range(5):
        for r in rot[i % 2]:
            x1 = (x1 + x2).astype(np.uint32)
            x2 = rotl(x2, r)
            x2 = (x2 ^ x1).astype(np.uint32)
        x1 = (x1 + ks[(i + 1) % 3]).astype(np.uint32)
        x2 = (x2 + ks[(i + 2) % 3] + np.uint32(i + 1)).astype(np.uint32)
    bits = (x1 ^ x2).astype(np.uint32)
    return ((bits >> np.uint32(9)) | np.uint32(0x3F800000)).view(
        np.float32) - np.float32(1.0)


@functools.cache
def _sample_consts():
    import numpy as np
    u_flat = _np_threefry_uniform(1, N_IMGS * S_PER_IMG)
    pad = N_RAYS - N_IMGS * S_PER_IMG
    u_flat = np.concatenate([u_flat, np.full((pad,), 0.5, np.float32)])
    offs = (np.minimum(np.arange(N_RAYS, dtype=np.int64) // S_PER_IMG,
                       N_IMGS - 1) * CDF_W).astype(np.int32)
    return u_flat, offs


def kernel(net_grad, loss_per_pix, prev_samples, cdf, noise, rand_ten,
           const_img_id):
    g = net_grad * 20.0 + noise * 0.02
    p = prev_samples + g
    q = p.at[-U_NUM:].set(rand_ten[-U_NUM:])

    loss2 = loss_per_pix.reshape(ROWS, LANES)
    p0 = p[:, 0].reshape(ROWS, LANES)
    p1 = p[:, 1].reshape(ROWS, LANES)
    id2 = const_img_id.astype(jnp.int32).reshape(ROWS, LANES)

    mask2, fi2 = _tc_stage(loss2, p0, p1, id2)
    q0_2 = q[:, 0]
    q1_2 = q[:, 1]

    u_flat, offs = _sample_consts()
    b1 = _sc_search_kernel()(jnp.asarray(u_flat), cdf.reshape(-1),
                             jnp.asarray(offs))
    o0, o1 = _sc_gather_kernel()(b1, fi2.reshape(-1), mask2.reshape(-1),
                                 q0_2.reshape(-1), q1_2.reshape(-1))
    return jnp.stack([o0, o1], axis=1)
